# baseline (speedup 1.0000x reference)
import functools

import jax
import jax.numpy as jnp
from jax import lax
from jax.experimental import pallas as pl
from jax.experimental.pallas import tpu as pltpu, tpu_sc as plsc

B, N, D = 32, 4096, 64
NC, NS, L = 2, 16, 16
NW = NC * NS
NVEC = N // L
IDX_PER_DMA = 128
ROWS_CHUNK = 512
N_DMA = ROWS_CHUNK // IDX_PER_DMA
N_CHUNKS = N // ROWS_CHUNK


def _sc_body(x_hbm, w_hbm, ll_hbm, pi_hbm, inds_hbm,
             mean_hbm, xr_hbm, wr_hbm, llr_hbm, pir_hbm,
             inds_v, w_v, ll_v, pi_v, flat_v,
             wr_v, llr_v, pir_v, wexp_v, rows_v, acc_v, sem):
  wid = lax.axis_index("s") * NC + lax.axis_index("c")
  base_row = wid * N

  pltpu.sync_copy(inds_hbm.at[wid], inds_v)
  pltpu.sync_copy(w_hbm.at[wid], w_v)
  pltpu.sync_copy(ll_hbm.at[wid], ll_v)
  pltpu.sync_copy(pi_hbm.at[wid], pi_v)

  def gather_body(j, mx):
    sl = pl.ds(j * L, L)
    idx16 = inds_v[sl]
    wr16 = plsc.load_gather(w_v, [idx16])
    wr_v[sl] = wr16
    llr_v[sl] = plsc.load_gather(ll_v, [idx16])
    pir_v[sl] = plsc.load_gather(pi_v, [idx16])
    flat_v[sl] = idx16 + base_row
    return jnp.maximum(mx, wr16)

  mx16 = lax.fori_loop(0, NVEC, gather_body,
                       jnp.full((L,), -jnp.inf, jnp.float32))
  m = lax.reduce_max_p.bind(mx16, axes=(0,))

  def exp_body(j, s):
    sl = pl.ds(j * L, L)
    e16 = jnp.exp(wr_v[sl] - m)
    wexp_v[sl] = e16
    return s + e16

  s16 = lax.fori_loop(0, NVEC, exp_body, jnp.zeros((L,), jnp.float32))
  z = lax.reduce_sum_p.bind(s16, axes=(0,))

  pltpu.sync_copy(wr_v, wr_hbm.at[wid])
  pltpu.sync_copy(llr_v, llr_hbm.at[wid])
  pltpu.sync_copy(pir_v, pir_hbm.at[wid])

  def chunk_body(c, acc):
    cbase = c * ROWS_CHUNK
    copies = []
    for t in range(N_DMA):
      idx_ref = flat_v.at[pl.ds(cbase + t * IDX_PER_DMA, IDX_PER_DMA)]
      dst = rows_v.at[pl.ds(t * IDX_PER_DMA, IDX_PER_DMA)]
      copies.append(pltpu.async_copy(x_hbm.at[idx_ref], dst, sem))
    for cp in copies:
      cp.wait()
    pltpu.sync_copy(rows_v, xr_hbm.at[pl.ds(base_row + cbase, ROWS_CHUNK)])

    def row_body(i, acc):
      bvec = plsc.load_gather(wexp_v, [jnp.full((L,), cbase + i, jnp.int32)])
      return tuple(acc[k] + bvec * rows_v[i, pl.ds(k * L, L)]
                   for k in range(D // L))

    return lax.fori_loop(0, ROWS_CHUNK, row_body, acc)

  acc0 = tuple(jnp.zeros((L,), jnp.float32) for _ in range(D // L))
  acc = lax.fori_loop(0, N_CHUNKS, chunk_body, acc0)

  z_vec = jnp.full((L,), z, jnp.float32)
  for k in range(D // L):
    acc_v[pl.ds(k * L, L)] = acc[k] / z_vec
  pltpu.sync_copy(acc_v, mean_hbm.at[wid])


@jax.jit
def kernel(x, w, ll, prev_inds, inds):
  xf = x.reshape(B * N, D)
  inds32 = inds.astype(jnp.int32)
  pi32 = prev_inds.astype(jnp.int32)

  mesh = plsc.VectorSubcoreMesh(core_axis_name="c", subcore_axis_name="s")
  run = pl.kernel(
      _sc_body,
      out_type=(
          jax.ShapeDtypeStruct((B, D), jnp.float32),
          jax.ShapeDtypeStruct((B * N, D), jnp.float32),
          jax.ShapeDtypeStruct((B, N), jnp.float32),
          jax.ShapeDtypeStruct((B, N), jnp.float32),
          jax.ShapeDtypeStruct((B, N), jnp.int32),
      ),
      mesh=mesh,
      compiler_params=pltpu.CompilerParams(needs_layout_passes=False,
                                           use_tc_tiling_on_sc=False),
      scratch_types=[
          pltpu.VMEM((N,), jnp.int32),
          pltpu.VMEM((N,), jnp.float32),
          pltpu.VMEM((N,), jnp.float32),
          pltpu.VMEM((N,), jnp.int32),
          pltpu.VMEM((N,), jnp.int32),
          pltpu.VMEM((N,), jnp.float32),
          pltpu.VMEM((N,), jnp.float32),
          pltpu.VMEM((N,), jnp.int32),
          pltpu.VMEM((N,), jnp.float32),
          pltpu.VMEM((ROWS_CHUNK, D), jnp.float32),
          pltpu.VMEM((D,), jnp.float32),
          pltpu.SemaphoreType.DMA,
      ],
  )
  mean, xr, wr, llr, pir = run(xf, w, ll, pi32, inds32)
  x_r = xr.reshape(B, N, D)
  return (mean, x_r, wr, llr, pir.astype(prev_inds.dtype))
